# Initial kernel scaffold; baseline (speedup 1.0000x reference)
import jax
import jax.numpy as jnp
from jax.experimental import pallas as pl


def kernel(images, proj_w, proj_b, ln_gamma, ln_beta):
    raise NotImplementedError("write your pallas kernel here")


# speedup vs baseline: 1.1167x; 1.1167x over previous
import jax
import jax.numpy as jnp
from jax.experimental import pallas as pl
from jax.experimental.pallas import tpu as pltpu

PATCH = 16
EMBED = 384
BINS = 32
LN_EPS = 1e-5
PPX = PATCH * PATCH


def _fused_kernel(img_ref, w_ref, b_ref, g_ref, bt_ref, x_ref, e_ref):
    img = img_ref[0]
    gH = img.shape[1] // PATCH
    gW = img.shape[2] // PATCH
    N = gH * gW

    pm = img.reshape(3, gH, PATCH, gW, PATCH)
    pm = pm.transpose(1, 3, 0, 2, 4).reshape(N, 3 * PPX)

    y = jnp.dot(pm, w_ref[...], preferred_element_type=jnp.float32) + b_ref[...]
    mu = jnp.mean(y, axis=1, keepdims=True)
    var = jnp.mean((y - mu) * (y - mu), axis=1, keepdims=True)
    x_ref[0] = (y - mu) * jax.lax.rsqrt(var + LN_EPS) * g_ref[...] + bt_ref[...]

    gray = (pm[:, :PPX] + pm[:, PPX:2 * PPX] + pm[:, 2 * PPX:]) / 3.0
    q = jnp.clip(gray * (BINS - 1), 0, BINS - 1).astype(jnp.int32)
    ent = jnp.zeros((N, 1), jnp.float32)
    inv_n = 1.0 / PPX
    for b in range(BINS):
        cb = jnp.sum(jnp.where(q == b, 1.0, 0.0), axis=1, keepdims=True)
        pb = cb * inv_n
        ent = ent - pb * jnp.log2(pb + 1e-10)
    scale = 1.0 / jnp.log2(jnp.float32(BINS))
    e_ref[...] = (ent * scale).reshape(1, 1, N)


def kernel(images, proj_w, proj_b, ln_gamma, ln_beta):
    B, C, H, W = images.shape
    p = PATCH
    gH, gW = H // p, W // p
    N = gH * gW
    K = C * PPX

    wmat = proj_w.reshape(EMBED, K).T

    x, ent = pl.pallas_call(
        _fused_kernel,
        grid=(B,),
        in_specs=[
            pl.BlockSpec((1, C, H, W), lambda i: (i, 0, 0, 0)),
            pl.BlockSpec((K, EMBED), lambda i: (0, 0)),
            pl.BlockSpec((1, EMBED), lambda i: (0, 0)),
            pl.BlockSpec((1, EMBED), lambda i: (0, 0)),
            pl.BlockSpec((1, EMBED), lambda i: (0, 0)),
        ],
        out_specs=[
            pl.BlockSpec((1, N, EMBED), lambda i: (i, 0, 0)),
            pl.BlockSpec((1, 1, N), lambda i: (i, 0, 0)),
        ],
        out_shape=[
            jax.ShapeDtypeStruct((B, N, EMBED), jnp.float32),
            jax.ShapeDtypeStruct((B, 1, N), jnp.float32),
        ],
        compiler_params=pltpu.CompilerParams(
            dimension_semantics=("parallel",),
        ),
    )(images, wmat, proj_b.reshape(1, EMBED),
      ln_gamma.reshape(1, EMBED), ln_beta.reshape(1, EMBED))

    return x, ent.reshape(B, N)


# speedup vs baseline: 1.1280x; 1.0102x over previous
import jax
import jax.numpy as jnp
from jax.experimental import pallas as pl
from jax.experimental.pallas import tpu as pltpu

PATCH = 16
EMBED = 384
BINS = 32
LN_EPS = 1e-5
PPX = PATCH * PATCH


def _fused_kernel(img_ref, w_ref, b_ref, g_ref, bt_ref, x_ref, e_ref):
    H = img_ref.shape[2]
    W = img_ref.shape[3]
    gH, gW = H // PATCH, W // PATCH
    N = gH * gW

    img = img_ref[0]

    gray = (img[0] + img[1] + img[2]) / 3.0
    q = jnp.clip(gray * (BINS - 1), 0, BINS - 1).astype(jnp.int32)
    qb = q.astype(jnp.bfloat16)

    stacked = jnp.concatenate([img.astype(jnp.bfloat16), qb[None]], axis=0)
    pm = stacked.reshape(4, gH, PATCH, gW, PATCH)
    pm = pm.transpose(1, 3, 0, 2, 4).reshape(N, 4 * PPX)

    y = jnp.dot(pm[:, :3 * PPX], w_ref[...],
                preferred_element_type=jnp.float32) + b_ref[...]
    mu = jnp.mean(y, axis=1, keepdims=True)
    var = jnp.mean((y - mu) * (y - mu), axis=1, keepdims=True)
    x_ref[0] = (y - mu) * jax.lax.rsqrt(var + LN_EPS) * g_ref[...] + bt_ref[...]

    qp = pm[:, 3 * PPX:]
    counts = [
        jnp.sum((qp == jnp.bfloat16(b)).astype(jnp.float32),
                axis=1, keepdims=True)
        for b in range(BINS)
    ]
    cnt = jnp.concatenate(counts, axis=1)
    pb = cnt * (1.0 / PPX)
    ent = -jnp.sum(pb * jnp.log2(pb + 1e-10), axis=1, keepdims=True)
    scale = 1.0 / jnp.log2(jnp.float32(BINS))
    e_ref[...] = (ent * scale).reshape(1, 1, N)


def kernel(images, proj_w, proj_b, ln_gamma, ln_beta):
    B, C, H, W = images.shape
    p = PATCH
    gH, gW = H // p, W // p
    N = gH * gW
    K = C * PPX

    wmat = proj_w.reshape(EMBED, K).T.astype(jnp.bfloat16)

    x, ent = pl.pallas_call(
        _fused_kernel,
        grid=(B,),
        in_specs=[
            pl.BlockSpec((1, C, H, W), lambda i: (i, 0, 0, 0)),
            pl.BlockSpec((K, EMBED), lambda i: (0, 0)),
            pl.BlockSpec((1, EMBED), lambda i: (0, 0)),
            pl.BlockSpec((1, EMBED), lambda i: (0, 0)),
            pl.BlockSpec((1, EMBED), lambda i: (0, 0)),
        ],
        out_specs=[
            pl.BlockSpec((1, N, EMBED), lambda i: (i, 0, 0)),
            pl.BlockSpec((1, 1, N), lambda i: (i, 0, 0)),
        ],
        out_shape=[
            jax.ShapeDtypeStruct((B, N, EMBED), jnp.float32),
            jax.ShapeDtypeStruct((B, 1, N), jnp.float32),
        ],
        compiler_params=pltpu.CompilerParams(
            dimension_semantics=("arbitrary",),
        ),
    )(images, wmat, proj_b.reshape(1, EMBED),
      ln_gamma.reshape(1, EMBED), ln_beta.reshape(1, EMBED))

    return x, ent.reshape(B, N)


# speedup vs baseline: 1.1299x; 1.0017x over previous
import jax
import jax.numpy as jnp
from jax.experimental import pallas as pl
from jax.experimental.pallas import tpu as pltpu

PATCH = 16
EMBED = 384
BINS = 32
LN_EPS = 1e-5
PPX = PATCH * PATCH


def _fused_kernel(img_ref, w_ref, b_ref, g_ref, bt_ref, x_ref, e_ref):
    IB = img_ref.shape[0]
    H = img_ref.shape[2]
    W = img_ref.shape[3]
    gH, gW = H // PATCH, W // PATCH
    N = gH * gW

    for i in range(IB):
        img = img_ref[i]

        gray = (img[0] + img[1] + img[2]) / 3.0
        q = jnp.clip(gray * (BINS - 1), 0, BINS - 1).astype(jnp.int32)
        qb = q.astype(jnp.bfloat16)

        stacked = jnp.concatenate([img.astype(jnp.bfloat16), qb[None]], axis=0)
        pm = stacked.reshape(4, gH, PATCH, gW, PATCH)
        pm = pm.transpose(1, 3, 0, 2, 4).reshape(N, 4 * PPX)

        y = jnp.dot(pm[:, :3 * PPX], w_ref[...],
                    preferred_element_type=jnp.float32) + b_ref[...]
        mu = jnp.mean(y, axis=1, keepdims=True)
        var = jnp.mean((y - mu) * (y - mu), axis=1, keepdims=True)
        x_ref[i] = (y - mu) * jax.lax.rsqrt(var + LN_EPS) * g_ref[...] + bt_ref[...]

        qp = pm[:, 3 * PPX:]
        counts = [
            jnp.sum((qp == jnp.bfloat16(b)).astype(jnp.float32),
                    axis=1, keepdims=True)
            for b in range(BINS)
        ]
        cnt = jnp.concatenate(counts, axis=1)
        pb = cnt * (1.0 / PPX)
        ent = -jnp.sum(pb * jnp.log2(pb + 1e-10), axis=1, keepdims=True)
        scale = 1.0 / jnp.log2(jnp.float32(BINS))
        e_ref[i] = (ent * scale).reshape(1, N)


def kernel(images, proj_w, proj_b, ln_gamma, ln_beta):
    B, C, H, W = images.shape
    p = PATCH
    gH, gW = H // p, W // p
    N = gH * gW
    K = C * PPX

    wmat = proj_w.reshape(EMBED, K).T.astype(jnp.bfloat16)

    IB = 2
    x, ent = pl.pallas_call(
        _fused_kernel,
        grid=(B // IB,),
        in_specs=[
            pl.BlockSpec((IB, C, H, W), lambda i: (i, 0, 0, 0)),
            pl.BlockSpec((K, EMBED), lambda i: (0, 0)),
            pl.BlockSpec((1, EMBED), lambda i: (0, 0)),
            pl.BlockSpec((1, EMBED), lambda i: (0, 0)),
            pl.BlockSpec((1, EMBED), lambda i: (0, 0)),
        ],
        out_specs=[
            pl.BlockSpec((IB, N, EMBED), lambda i: (i, 0, 0)),
            pl.BlockSpec((IB, 1, N), lambda i: (i, 0, 0)),
        ],
        out_shape=[
            jax.ShapeDtypeStruct((B, N, EMBED), jnp.float32),
            jax.ShapeDtypeStruct((B, 1, N), jnp.float32),
        ],
        compiler_params=pltpu.CompilerParams(
            dimension_semantics=("arbitrary",),
        ),
    )(images, wmat, proj_b.reshape(1, EMBED),
      ln_gamma.reshape(1, EMBED), ln_beta.reshape(1, EMBED))

    return x, ent.reshape(B, N)
